# baseline (speedup 1.0000x reference)
import functools
import math

import jax
import jax.numpy as jnp
from jax import lax
from jax.experimental import pallas as pl
from jax.experimental.pallas import tpu as pltpu
from jax.experimental.pallas import tpu_sc as plsc

D_MODEL = 768
EMB_DIM = 64
SCALE = math.sqrt(D_MODEL)

_info = plsc.get_sparse_core_info()
_NC = _info.num_cores
_NS = _info.num_subcores
_NW = _NC * _NS


@functools.cache
def _make_gather(V, D, B):
    assert B % (8 * _NW) == 0 and D % 16 == 0
    b_per_w = B // _NW
    mesh = plsc.VectorSubcoreMesh(core_axis_name="c", subcore_axis_name="s")

    @functools.partial(
        pl.kernel,
        mesh=mesh,
        out_type=jax.ShapeDtypeStruct((B, D), jnp.float32),
        scratch_types=[
            pltpu.VMEM((b_per_w,), jnp.int32),
            pltpu.VMEM((b_per_w, D), jnp.float32),
            pltpu.SemaphoreType.DMA,
        ],
        compiler_params=pltpu.CompilerParams(use_tc_tiling_on_sc=False),
    )
    def gather_k(table_hbm, idx_hbm, out_hbm, idx_v, rows_v, sem):
        wid = lax.axis_index("s") * _NC + lax.axis_index("c")
        base = wid * b_per_w
        pltpu.sync_copy(idx_hbm.at[pl.ds(base, b_per_w)], idx_v)
        pltpu.async_copy(table_hbm.at[idx_v], rows_v, sem).wait()
        pltpu.sync_copy(rows_v, out_hbm.at[pl.ds(base, b_per_w)])

    return gather_k


def _proj_body(emb_ref, wt_ref, b_ref, out_ref):
    acc = jnp.dot(emb_ref[...], wt_ref[...], preferred_element_type=jnp.float32)
    out_ref[...] = (acc + b_ref[...]) * SCALE


@functools.cache
def _make_proj(B, blk):
    grid = (B // blk,)
    return pl.pallas_call(
        _proj_body,
        grid=grid,
        in_specs=[
            pl.BlockSpec((blk, EMB_DIM), lambda i: (i, 0)),
            pl.BlockSpec((EMB_DIM, D_MODEL), lambda i: (0, 0)),
            pl.BlockSpec((1, D_MODEL), lambda i: (0, 0)),
        ],
        out_specs=pl.BlockSpec((blk, D_MODEL), lambda i: (i, 0)),
        out_shape=jax.ShapeDtypeStruct((B, D_MODEL), jnp.float32),
    )


def kernel(x, table, W, b):
    Bo, L = x.shape
    V, D = table.shape
    B = Bo * L
    idx = x.reshape(B).astype(jnp.int32)
    gathered = _make_gather(V, D, B)(table, idx)
    out = _make_proj(B, 1024)(gathered, W.T, b.reshape(1, D_MODEL))
    return out.reshape(Bo, L, D_MODEL).astype(jnp.float32)


# speedup vs baseline: 1.4537x; 1.4537x over previous
import functools
import math

import jax
import jax.numpy as jnp
from jax import lax
from jax.experimental import pallas as pl
from jax.experimental.pallas import tpu as pltpu
from jax.experimental.pallas import tpu_sc as plsc

D_MODEL = 768
EMB_DIM = 64
SCALE = math.sqrt(D_MODEL)

_info = plsc.get_sparse_core_info()
_NC = _info.num_cores
_NS = _info.num_subcores
_NW = _NC * _NS

_CH = 16


@functools.cache
def _make_gather(V, D, B):
    assert B % (16 * _NW) == 0 and D == EMB_DIM
    b_per_w = B // _NW
    mesh = plsc.VectorSubcoreMesh(core_axis_name="c", subcore_axis_name="s")

    @functools.partial(
        pl.kernel,
        mesh=mesh,
        out_type=jax.ShapeDtypeStruct((B, 128), jnp.float32),
        scratch_types=[
            pltpu.VMEM((b_per_w,), jnp.int32),
            pltpu.VMEM((b_per_w, 128), jnp.float32),
            pltpu.SemaphoreType.DMA,
        ],
    )
    def gather_k(table_hbm, idx_hbm, out_hbm, idx_v, rows_v, sem):
        wid = lax.axis_index("s") * _NC + lax.axis_index("c")
        base = wid * b_per_w
        pltpu.sync_copy(idx_hbm.at[pl.ds(base, b_per_w)], idx_v)

        def chunk(c, carry):
            i0 = c * _CH
            vec = idx_v[pl.ds(i0, 16)]
            for j in range(_CH):
                r = vec[j]
                pltpu.async_copy(
                    table_hbm.at[r], rows_v.at[i0 + j, pl.ds(0, D)], sem
                )
            for j in range(_CH):
                pltpu.make_async_copy(
                    table_hbm.at[0], rows_v.at[i0 + j, pl.ds(0, D)], sem
                ).wait()
            return carry

        lax.fori_loop(0, b_per_w // _CH, chunk, 0)
        pltpu.sync_copy(rows_v, out_hbm.at[pl.ds(base, b_per_w)])

    return gather_k


def _proj_body(emb_ref, wt_ref, b_ref, out_ref):
    acc = jnp.dot(
        emb_ref[:, :EMB_DIM], wt_ref[...], preferred_element_type=jnp.float32
    )
    out_ref[...] = (acc + b_ref[...]) * SCALE


@functools.cache
def _make_proj(B, blk):
    grid = (B // blk,)
    return pl.pallas_call(
        _proj_body,
        grid=grid,
        in_specs=[
            pl.BlockSpec((blk, 128), lambda i: (i, 0)),
            pl.BlockSpec((EMB_DIM, D_MODEL), lambda i: (0, 0)),
            pl.BlockSpec((1, D_MODEL), lambda i: (0, 0)),
        ],
        out_specs=pl.BlockSpec((blk, D_MODEL), lambda i: (i, 0)),
        out_shape=jax.ShapeDtypeStruct((B, D_MODEL), jnp.float32),
    )


def kernel(x, table, W, b):
    Bo, L = x.shape
    V, D = table.shape
    B = Bo * L
    idx = x.reshape(B).astype(jnp.int32)
    gathered = _make_gather(V, D, B)(table, idx)
    out = _make_proj(B, 1024)(gathered, W.T, b.reshape(1, D_MODEL))
    return out.reshape(Bo, L, D_MODEL).astype(jnp.float32)


# speedup vs baseline: 1.5119x; 1.0400x over previous
import functools
import math

import jax
import jax.numpy as jnp
from jax import lax
from jax.experimental import pallas as pl
from jax.experimental.pallas import tpu as pltpu
from jax.experimental.pallas import tpu_sc as plsc

D_MODEL = 768
EMB_DIM = 64
SCALE = math.sqrt(D_MODEL)

_info = plsc.get_sparse_core_info()
_NC = _info.num_cores
_NS = _info.num_subcores
_NW = _NC * _NS

_CH = 64


@functools.cache
def _make_gather(V, D, B):
    assert B % (16 * _NW) == 0 and D == EMB_DIM
    b_per_w = B // _NW
    mesh = plsc.VectorSubcoreMesh(core_axis_name="c", subcore_axis_name="s")

    @functools.partial(
        pl.kernel,
        mesh=mesh,
        out_type=jax.ShapeDtypeStruct((B, 128), jnp.float32),
        scratch_types=[
            pltpu.VMEM((b_per_w,), jnp.int32),
            pltpu.VMEM((b_per_w, 128), jnp.float32),
            pltpu.SemaphoreType.DMA,
        ],
    )
    def gather_k(table_hbm, idx_hbm, out_hbm, idx_v, rows_v, sem):
        wid = lax.axis_index("s") * _NC + lax.axis_index("c")
        base = wid * b_per_w
        pltpu.sync_copy(idx_hbm.at[pl.ds(base, b_per_w)], idx_v)

        def fire(i0):
            for g in range(_CH // 16):
                vec = idx_v[pl.ds(i0 + g * 16, 16)]
                for j in range(16):
                    pltpu.async_copy(
                        table_hbm.at[vec[j]],
                        rows_v.at[i0 + g * 16 + j, pl.ds(0, D)],
                        sem,
                    )

        def drain(i0):
            for j in range(_CH):
                pltpu.make_async_copy(
                    table_hbm.at[0], rows_v.at[i0 + j, pl.ds(0, D)], sem
                ).wait()

        n_chunks = b_per_w // _CH
        fire(0)

        def chunk(c, carry):
            fire((c + 1) * _CH)
            drain(c * _CH)
            return carry

        lax.fori_loop(0, n_chunks - 1, chunk, 0)
        drain((n_chunks - 1) * _CH)
        pltpu.sync_copy(rows_v, out_hbm.at[pl.ds(base, b_per_w)])

    return gather_k


def _proj_body(emb_ref, wt_ref, b_ref, out_ref):
    acc = jnp.dot(
        emb_ref[:, :EMB_DIM], wt_ref[...], preferred_element_type=jnp.float32
    )
    out_ref[...] = (acc + b_ref[...]) * SCALE


@functools.cache
def _make_proj(B, blk):
    grid = (B // blk,)
    return pl.pallas_call(
        _proj_body,
        grid=grid,
        in_specs=[
            pl.BlockSpec((blk, 128), lambda i: (i, 0)),
            pl.BlockSpec((EMB_DIM, D_MODEL), lambda i: (0, 0)),
            pl.BlockSpec((1, D_MODEL), lambda i: (0, 0)),
        ],
        out_specs=pl.BlockSpec((blk, D_MODEL), lambda i: (i, 0)),
        out_shape=jax.ShapeDtypeStruct((B, D_MODEL), jnp.float32),
    )


def kernel(x, table, W, b):
    Bo, L = x.shape
    V, D = table.shape
    B = Bo * L
    idx = x.reshape(B).astype(jnp.int32)
    gathered = _make_gather(V, D, B)(table, idx)
    out = _make_proj(B, 1024)(gathered, W.T, b.reshape(1, D_MODEL))
    return out.reshape(Bo, L, D_MODEL).astype(jnp.float32)


# speedup vs baseline: 1.5119x; 1.0000x over previous
import functools
import math

import jax
import jax.numpy as jnp
from jax import lax
from jax.experimental import pallas as pl
from jax.experimental.pallas import tpu as pltpu
from jax.experimental.pallas import tpu_sc as plsc

D_MODEL = 768
EMB_DIM = 64
SCALE = math.sqrt(D_MODEL)

_info = plsc.get_sparse_core_info()
_NC = _info.num_cores
_NS = _info.num_subcores
_NW = _NC * _NS

_CH = 64


@functools.cache
def _make_gather(V, D, B):
    assert B % (16 * _NW) == 0 and D == EMB_DIM
    b_per_w = B // _NW
    mesh = plsc.VectorSubcoreMesh(core_axis_name="c", subcore_axis_name="s")

    @functools.partial(
        pl.kernel,
        mesh=mesh,
        out_type=jax.ShapeDtypeStruct((B, 128), jnp.float32),
        scratch_types=[
            pltpu.VMEM((b_per_w,), jnp.int32),
            pltpu.VMEM((b_per_w, 128), jnp.float32),
            pltpu.SemaphoreType.DMA,
        ],
        compiler_params=pltpu.CompilerParams(skip_device_barrier=True),
    )
    def gather_k(table_hbm, idx_hbm, out_hbm, idx_v, rows_v, sem):
        wid = lax.axis_index("s") * _NC + lax.axis_index("c")
        base = wid * b_per_w
        pltpu.sync_copy(idx_hbm.at[pl.ds(base, b_per_w)], idx_v)

        def fire(i0):
            for g in range(_CH // 16):
                vec = idx_v[pl.ds(i0 + g * 16, 16)]
                for j in range(16):
                    pltpu.async_copy(
                        table_hbm.at[vec[j]],
                        rows_v.at[i0 + g * 16 + j, pl.ds(0, D)],
                        sem,
                    )

        def drain(i0):
            for j in range(_CH):
                pltpu.make_async_copy(
                    table_hbm.at[0], rows_v.at[i0 + j, pl.ds(0, D)], sem
                ).wait()

        if True:
            n_chunks = b_per_w // _CH
            fire(0)

            def chunk(c, carry):
                fire((c + 1) * _CH)
                drain(c * _CH)
                return carry

            lax.fori_loop(0, n_chunks - 1, chunk, 0)
            drain((n_chunks - 1) * _CH)
        pltpu.sync_copy(rows_v, out_hbm.at[pl.ds(base, b_per_w)])

    return gather_k


def _proj_body(emb_ref, wt_ref, b_ref, out_ref):
    acc = jnp.dot(
        emb_ref[:, :EMB_DIM], wt_ref[...], preferred_element_type=jnp.float32
    )
    out_ref[...] = (acc + b_ref[...]) * SCALE


@functools.cache
def _make_proj(B, blk):
    grid = (B // blk,)
    return pl.pallas_call(
        _proj_body,
        grid=grid,
        in_specs=[
            pl.BlockSpec((blk, 128), lambda i: (i, 0)),
            pl.BlockSpec((EMB_DIM, D_MODEL), lambda i: (0, 0)),
            pl.BlockSpec((1, D_MODEL), lambda i: (0, 0)),
        ],
        out_specs=pl.BlockSpec((blk, D_MODEL), lambda i: (i, 0)),
        out_shape=jax.ShapeDtypeStruct((B, D_MODEL), jnp.float32),
    )


def kernel(x, table, W, b):
    Bo, L = x.shape
    V, D = table.shape
    B = Bo * L
    idx = x.reshape(B).astype(jnp.int32)
    gathered = _make_gather(V, D, B)(table, idx)
    out = _make_proj(B, 1024)(gathered, W.T, b.reshape(1, D_MODEL))
    return out.reshape(Bo, L, D_MODEL).astype(jnp.float32)


# speedup vs baseline: 1.7678x; 1.1692x over previous
import functools
import math

import jax
import jax.numpy as jnp
from jax import lax
from jax.experimental import pallas as pl
from jax.experimental.pallas import tpu as pltpu
from jax.experimental.pallas import tpu_sc as plsc

D_MODEL = 768
EMB_DIM = 64
SCALE = math.sqrt(D_MODEL)

_info = plsc.get_sparse_core_info()
_NC = _info.num_cores
_NS = _info.num_subcores
_NW = _NC * _NS

_CH = 64


@functools.cache
def _make_gather(V, D, B):
    assert B % (16 * _NW) == 0 and D == EMB_DIM
    b_per_w = B // _NW
    mesh = plsc.VectorSubcoreMesh(core_axis_name="c", subcore_axis_name="s")

    @functools.partial(
        pl.kernel,
        mesh=mesh,
        out_type=jax.ShapeDtypeStruct((B, 128), jnp.float32),
        scratch_types=[
            pltpu.VMEM((b_per_w,), jnp.int32),
            pltpu.VMEM((b_per_w, 128), jnp.float32),
            pltpu.SemaphoreType.DMA,
        ],
        compiler_params=pltpu.CompilerParams(skip_device_barrier=True),
    )
    def gather_k(table_hbm, idx_hbm, out_hbm, idx_v, rows_v, sem):
        wid = lax.axis_index("s") * _NC + lax.axis_index("c")
        base = wid * b_per_w
        pltpu.sync_copy(idx_hbm.at[pl.ds(base, b_per_w)], idx_v)

        def fire(i0):
            for g in range(_CH // 16):
                vec = idx_v[pl.ds(i0 + g * 16, 16)]
                for j in range(16):
                    pltpu.async_copy(
                        table_hbm.at[vec[j]],
                        rows_v.at[i0 + g * 16 + j, pl.ds(0, D)],
                        sem,
                    )

        def drain(i0):
            for j in range(_CH):
                pltpu.make_async_copy(
                    table_hbm.at[0], rows_v.at[i0 + j, pl.ds(0, D)], sem
                ).wait()

        if True:
            n_chunks = b_per_w // _CH
            fire(0)

            def chunk(c, carry):
                fire((c + 1) * _CH)
                drain(c * _CH)
                return carry

            lax.fori_loop(0, n_chunks - 1, chunk, 0)
            drain((n_chunks - 1) * _CH)
        pltpu.sync_copy(rows_v, out_hbm.at[pl.ds(base, b_per_w)])

    return gather_k


@functools.cache
def _make_proj(Bo, L, bb):
    def body(emb_ref, wt_ref, b_ref, out_ref):
        acc = jnp.dot(
            emb_ref[:, :EMB_DIM], wt_ref[...], preferred_element_type=jnp.float32
        )
        acc = (acc + b_ref[...]) * SCALE
        out_ref[...] = acc.reshape(bb, L, D_MODEL)

    return pl.pallas_call(
        body,
        grid=(Bo // bb,),
        in_specs=[
            pl.BlockSpec((bb * L, 128), lambda i: (i, 0)),
            pl.BlockSpec((EMB_DIM, D_MODEL), lambda i: (0, 0)),
            pl.BlockSpec((1, D_MODEL), lambda i: (0, 0)),
        ],
        out_specs=pl.BlockSpec((bb, L, D_MODEL), lambda i: (i, 0, 0)),
        out_shape=jax.ShapeDtypeStruct((Bo, L, D_MODEL), jnp.float32),
    )


def kernel(x, table, W, b):
    Bo, L = x.shape
    V, D = table.shape
    B = Bo * L
    idx = x.reshape(B).astype(jnp.int32)
    gathered = _make_gather(V, D, B)(table, idx)
    return _make_proj(Bo, L, 64)(gathered, W.T, b.reshape(1, D_MODEL))


# speedup vs baseline: 1.9676x; 1.1130x over previous
import functools
import math

import jax
import jax.numpy as jnp
from jax import lax
from jax.experimental import pallas as pl
from jax.experimental.pallas import tpu as pltpu
from jax.experimental.pallas import tpu_sc as plsc

D_MODEL = 768
EMB_DIM = 64
SCALE = math.sqrt(D_MODEL)

_info = plsc.get_sparse_core_info()
_NC = _info.num_cores
_NS = _info.num_subcores
_NW = _NC * _NS

_CH = 64


@functools.cache
def _make_gather(V, D, B):
    assert B % (16 * _NW) == 0 and D == EMB_DIM
    b_per_w = B // _NW
    mesh = plsc.VectorSubcoreMesh(core_axis_name="c", subcore_axis_name="s")

    @functools.partial(
        pl.kernel,
        mesh=mesh,
        out_type=jax.ShapeDtypeStruct((B, 128), jnp.float32),
        scratch_types=[
            pltpu.VMEM((b_per_w,), jnp.int32),
            pltpu.VMEM((b_per_w, 128), jnp.float32),
            pltpu.SemaphoreType.DMA,
        ],
        compiler_params=pltpu.CompilerParams(skip_device_barrier=True),
    )
    def gather_k(table_hbm, idx_hbm, out_hbm, idx_v, rows_v, sem):
        wid = lax.axis_index("s") * _NC + lax.axis_index("c")
        base = wid * b_per_w
        pltpu.sync_copy(idx_hbm.at[pl.ds(base, b_per_w)], idx_v)

        def fire(i0):
            for g in range(_CH // 16):
                vec = idx_v[pl.ds(i0 + g * 16, 16)]
                for j in range(16):
                    pltpu.async_copy(
                        table_hbm.at[vec[j]],
                        rows_v.at[i0 + g * 16 + j, pl.ds(0, D)],
                        sem,
                    )

        def drain(i0):
            for j in range(_CH):
                pltpu.make_async_copy(
                    table_hbm.at[0], rows_v.at[i0 + j, pl.ds(0, D)], sem
                ).wait()

        if True:
            n_chunks = b_per_w // _CH
            fire(0)

            def chunk(c, carry):
                fire((c + 1) * _CH)
                drain(c * _CH)
                return carry

            lax.fori_loop(0, n_chunks - 1, chunk, 0)
            drain((n_chunks - 1) * _CH)
        pltpu.sync_copy(rows_v, out_hbm.at[pl.ds(base, b_per_w)])

    return gather_k


@functools.cache
def _make_proj(Bo, L):
    def body(emb_ref, wt_ref, b_ref, out_ref):
        acc = jnp.dot(
            emb_ref[:, :EMB_DIM], wt_ref[...], preferred_element_type=jnp.float32
        )
        acc = (acc + b_ref[...]) * SCALE
        out_ref[...] = acc.reshape(1, Bo, D_MODEL)

    return pl.pallas_call(
        body,
        grid=(L,),
        in_specs=[
            pl.BlockSpec((Bo, 128), lambda i: (i, 0)),
            pl.BlockSpec((EMB_DIM, D_MODEL), lambda i: (0, 0)),
            pl.BlockSpec((1, D_MODEL), lambda i: (0, 0)),
        ],
        out_specs=pl.BlockSpec((1, Bo, D_MODEL), lambda i: (i, 0, 0)),
        out_shape=jax.ShapeDtypeStruct((L, Bo, D_MODEL), jnp.float32),
    )


def kernel(x, table, W, b):
    Bo, L = x.shape
    V, D = table.shape
    B = Bo * L
    idx = jnp.transpose(x).reshape(B).astype(jnp.int32)
    gathered = _make_gather(V, D, B)(table, idx)
    out = _make_proj(Bo, L)(gathered, W.T, b.reshape(1, D_MODEL))
    return jnp.transpose(out, (1, 0, 2))


# speedup vs baseline: 1.9830x; 1.0079x over previous
import functools
import math

import jax
import jax.numpy as jnp
from jax import lax
from jax.experimental import pallas as pl
from jax.experimental.pallas import tpu as pltpu
from jax.experimental.pallas import tpu_sc as plsc

D_MODEL = 768
EMB_DIM = 64
SCALE = math.sqrt(D_MODEL)

_info = plsc.get_sparse_core_info()
_NC = _info.num_cores
_NS = _info.num_subcores
_NW = _NC * _NS

_CH = 64


@functools.cache
def _make_gather(V, D, B):
    assert B % (16 * _NW) == 0 and D == EMB_DIM
    b_per_w = B // _NW
    mesh = plsc.VectorSubcoreMesh(core_axis_name="c", subcore_axis_name="s")

    @functools.partial(
        pl.kernel,
        mesh=mesh,
        out_type=jax.ShapeDtypeStruct((B, 128), jnp.float32),
        scratch_types=[
            pltpu.VMEM((b_per_w,), jnp.int32),
            pltpu.VMEM((b_per_w, 128), jnp.float32),
            pltpu.SemaphoreType.DMA,
        ],
    )
    def gather_k(table_hbm, idx_hbm, out_hbm, idx_v, rows_v, sem):
        wid = lax.axis_index("s") * _NC + lax.axis_index("c")
        base = wid * b_per_w
        pltpu.sync_copy(idx_hbm.at[pl.ds(base, b_per_w)], idx_v)

        def fire(i0):
            for g in range(_CH // 16):
                vec = idx_v[pl.ds(i0 + g * 16, 16)]
                for j in range(16):
                    pltpu.async_copy(
                        table_hbm.at[vec[j]],
                        rows_v.at[i0 + g * 16 + j, pl.ds(0, D)],
                        sem,
                    )

        def drain(i0):
            for j in range(_CH):
                pltpu.make_async_copy(
                    table_hbm.at[0], rows_v.at[i0 + j, pl.ds(0, D)], sem
                ).wait()

        n_chunks = b_per_w // _CH
        fire(0)

        def chunk(c, carry):
            fire((c + 1) * _CH)
            drain(c * _CH)
            return carry

        lax.fori_loop(0, n_chunks - 1, chunk, 0)
        drain((n_chunks - 1) * _CH)
        pltpu.sync_copy(rows_v, out_hbm.at[pl.ds(base, b_per_w)])

    return gather_k


@functools.cache
def _make_proj(Bo, L):
    def body(emb_ref, wt_ref, b_ref, out_ref):
        acc = jnp.dot(
            emb_ref[:, :EMB_DIM], wt_ref[...], preferred_element_type=jnp.float32
        )
        acc = (acc + b_ref[...]) * SCALE
        out_ref[...] = acc.reshape(1, Bo, D_MODEL)

    return pl.pallas_call(
        body,
        grid=(L,),
        in_specs=[
            pl.BlockSpec((Bo, 128), lambda i: (i, 0)),
            pl.BlockSpec((EMB_DIM, D_MODEL), lambda i: (0, 0)),
            pl.BlockSpec((1, D_MODEL), lambda i: (0, 0)),
        ],
        out_specs=pl.BlockSpec((1, Bo, D_MODEL), lambda i: (i, 0, 0)),
        out_shape=jax.ShapeDtypeStruct((L, Bo, D_MODEL), jnp.float32),
    )


def kernel(x, table, W, b):
    Bo, L = x.shape
    V, D = table.shape
    B = Bo * L
    idx = jnp.transpose(x).reshape(B).astype(jnp.int32)
    gathered = _make_gather(V, D, B)(table, idx)
    out = _make_proj(Bo, L)(gathered, W.T, b.reshape(1, D_MODEL))
    return jnp.transpose(out, (1, 0, 2))
